# baseline (speedup 1.0000x reference)
import functools

import jax
import jax.numpy as jnp
from jax import lax
from jax.experimental import pallas as pl
from jax.experimental.pallas import tpu as pltpu
from jax.experimental.pallas import tpu_sc as plsc

_BW = 512
_HB = 128


def _build(nb, nf, d, nc, ns):
    mesh = plsc.VectorSubcoreMesh(core_axis_name="c", subcore_axis_name="s")
    pack = 128 // d
    nh = _BW // _HB
    nu = nf * nh

    @functools.partial(
        pl.kernel,
        out_type=jax.ShapeDtypeStruct((nf, d, nb), jnp.float32),
        mesh=mesh,
        scratch_types=[
            pltpu.VMEM((nf, _BW), jnp.int32),
            pltpu.VMEM((nu, _HB), jnp.int32),
            pltpu.VMEM((_HB, 128), jnp.float32),
            pltpu.VMEM((_HB, 128), jnp.float32),
            pltpu.VMEM((d, _HB), jnp.float32),
            pltpu.VMEM((d, _HB), jnp.float32),
            pltpu.SemaphoreType.DMA,
            pltpu.SemaphoreType.DMA,
            pltpu.SemaphoreType.DMA,
            pltpu.SemaphoreType.DMA,
        ],
        compiler_params=pltpu.CompilerParams(
            use_tc_tiling_on_sc=True, needs_layout_passes=False),
    )
    def run(idx_hbm, table_hbm, out_hbm, idx_v, idxs_v,
            a0, a1, b0, b1, g0, g1, w0, w1):
        wid = lax.axis_index("s") * nc + lax.axis_index("c")
        base = wid * _BW
        pltpu.sync_copy(idx_hbm.at[:, pl.ds(base, _BW)], idx_v)
        iota = lax.iota(jnp.int32, 16)

        def shift_f(f, carry):
            def shift_h(h, c1):
                def shift_q(q, c2):
                    v = idx_v[f, pl.ds(h * _HB + q * 16, 16)]
                    idxs_v[f * nh + h, pl.ds(q * 16, 16)] = (
                        lax.shift_right_logical(v, 2))
                    return c2
                lax.fori_loop(0, _HB // 16, shift_q, 0)
                return c1
            lax.fori_loop(0, nh, shift_h, 0)
            return carry

        lax.fori_loop(0, nf, shift_f, 0)

        def gather(u, a, sem):
            return pltpu.async_copy(table_hbm.at[idxs_v.at[u]], a, sem)

        def transpose(f, h, a, b):
            for q in range(_HB // 16):
                raw = idx_v[f, pl.ds(h * _HB + q * 16, 16)]
                cbase = (raw & (pack - 1)) * d
                row = q * 16 + iota
                for dd in range(d):
                    v = plsc.load_gather(a, [row, cbase + dd])
                    b[dd, pl.ds(q * 16, 16)] = v

        def write(f, h, b, sem):
            return pltpu.async_copy(
                b, out_hbm.at[f, :, pl.ds(base + h * _HB, _HB)], sem)

        def body(i, carry):
            u0 = 2 * i
            u1 = u0 + 1
            f0, h0 = u0 // nh, u0 % nh
            f1, h1 = u1 // nh, u1 % nh
            ga0 = gather(u0, a0, g0)
            ga1 = gather(u1, a1, g1)
            ga0.wait()
            transpose(f0, h0, a0, b0)
            wb0 = write(f0, h0, b0, w0)
            ga1.wait()
            transpose(f1, h1, a1, b1)
            wb1 = write(f1, h1, b1, w1)
            wb0.wait()
            wb1.wait()
            return carry

        lax.fori_loop(0, nu // 2, body, 0)

    return run


def kernel(indices, weight):
    nb, nf = indices.shape
    v, d = weight.shape
    info = plsc.get_sparse_core_info()
    idx_t = indices.T.astype(jnp.int32)
    table = weight.reshape(v * d // 128, 128)
    run = _build(nb, nf, d, info.num_cores, info.num_subcores)
    out_t = run(idx_t, table)
    return out_t.transpose(2, 0, 1)
